# baseline (speedup 1.0000x reference)
import functools

import jax
import jax.numpy as jnp
from jax import lax
from jax.experimental import pallas as pl
from jax.experimental.pallas import tpu as pltpu
from jax.experimental.pallas import tpu_sc as plsc

N = 16384
D_DENSE = 2048
D_SPARSE = 1024
E = 64
TOP_K = 8
BN = 2048

_NC, _NS, _L = 2, 16, 16
_NW = _NC * _NS
_ROWS = N // _NW


def _tc_body(d_ref, s_ref, wd_ref, bd_ref, ws_ref, bs_ref, b_ref,
             logits_ref, cd_ref, cs_ref):
    cd = jnp.dot(d_ref[...], wd_ref[...],
                 preferred_element_type=jnp.float32) + bd_ref[...]
    cs = jnp.dot(s_ref[...], ws_ref[...],
                 preferred_element_type=jnp.float32) + bs_ref[...]
    cd_ref[...] = cd
    cs_ref[...] = cs
    logits_ref[...] = b_ref[...] + cd + cs


def _sc_body(logits_hbm, mask_hbm, w_hbm, lg_v, mk_v, wt_v):
    wid = lax.axis_index("s") * _NC + lax.axis_index("c")
    base = wid * _ROWS * E
    pltpu.sync_copy(logits_hbm.at[pl.ds(base, _ROWS * E)], lg_v)

    lane = lax.iota(jnp.int32, _L)
    perm_hi = (lane + 8) & 15

    def merge_top8(a, b):
        b_perm = b.at[perm_hi].get(mode="promise_in_bounds")
        comb = jnp.where(lane < 8, a, b_perm)
        s, _ = plsc.sort_key_val(comb, lane, descending=True)
        return s

    def tok(t, carry):
        v = [lg_v[pl.ds(t * E + _L * i, _L)] for i in range(4)]
        s = [plsc.sort_key_val(v[i], lane, descending=True)[0]
             for i in range(4)]
        fin = merge_top8(merge_top8(s[0], s[1]), merge_top8(s[2], s[3]))
        t8 = fin.at[lane * 0 + (TOP_K - 1)].get(mode="promise_in_bounds")
        m0 = fin.at[lane * 0].get(mode="promise_in_bounds")
        es = []
        for i in range(4):
            ge = v[i] >= t8
            mk_v[pl.ds(t * E + _L * i, _L)] = jnp.where(ge, 1.0, 0.0)
            es.append(jnp.where(ge, jnp.exp(v[i] - m0), 0.0))
        tot = jnp.sum(es[0] + es[1] + es[2] + es[3])
        for i in range(4):
            wt_v[pl.ds(t * E + _L * i, _L)] = es[i] / tot
        return carry

    lax.fori_loop(0, _ROWS, tok, 0)
    pltpu.sync_copy(mk_v, mask_hbm.at[pl.ds(base, _ROWS * E)])
    pltpu.sync_copy(wt_v, w_hbm.at[pl.ds(base, _ROWS * E)])


def _sc_route(logits):
    f = pl.kernel(
        _sc_body,
        out_type=[jax.ShapeDtypeStruct((N * E,), jnp.float32)] * 2,
        mesh=plsc.VectorSubcoreMesh(
            core_axis_name="c", subcore_axis_name="s",
            num_cores=_NC, num_subcores=_NS),
        scratch_types=[pltpu.VMEM((_ROWS * E,), jnp.float32)] * 3,
        compiler_params=pltpu.CompilerParams(needs_layout_passes=False),
    )
    mask, w = f(logits.reshape(N * E))
    return mask.reshape(N, E), w.reshape(N, E)


@jax.jit
def _router(dense, sparse, W_dense, b_dense, W_sparse, b_sparse, bias):
    grid = (N // BN,)
    row_spec = pl.BlockSpec((BN, E), lambda i: (i, 0))
    full = lambda shape: pl.BlockSpec(shape, lambda i: (0, 0))
    logits, cd, cs = pl.pallas_call(
        _tc_body,
        grid=grid,
        in_specs=[
            pl.BlockSpec((BN, D_DENSE), lambda i: (i, 0)),
            pl.BlockSpec((BN, D_SPARSE), lambda i: (i, 0)),
            full((D_DENSE, E)),
            full((1, E)),
            full((D_SPARSE, E)),
            full((1, E)),
            full((1, E)),
        ],
        out_specs=[row_spec] * 3,
        out_shape=[jax.ShapeDtypeStruct((N, E), jnp.float32)] * 3,
    )(dense, sparse, W_dense, b_dense.reshape(1, E),
      W_sparse, b_sparse.reshape(1, E), bias.reshape(1, E))
    mask, weights = _sc_route(logits)
    return logits, weights, mask, cd, cs


def kernel(dense, sparse, W_dense, b_dense, W_sparse, b_sparse, bias):
    logits, weights, topk_mask, c_dense, c_sparse = _router(
        dense, sparse, W_dense, b_dense, W_sparse, b_sparse, bias)
    return (logits, weights, topk_mask, c_dense, c_sparse)


# speedup vs baseline: 1.0048x; 1.0048x over previous
import functools

import jax
import jax.numpy as jnp
from jax import lax
from jax.experimental import pallas as pl
from jax.experimental.pallas import tpu as pltpu
from jax.experimental.pallas import tpu_sc as plsc

N = 16384
D_DENSE = 2048
D_SPARSE = 1024
E = 64
TOP_K = 8
BN = 2048

_NC, _NS, _L = 2, 16, 16
_NW = _NC * _NS
_ROWS = N // _NW


def _tc_body(d0_ref, d1_ref, d2_ref, d3_ref, s0_ref, s1_ref,
             wd0_ref, wd1_ref, wd2_ref, wd3_ref, bd_ref,
             ws0_ref, ws1_ref, bs_ref, b_ref,
             logits_ref, cd_ref, cs_ref):
    cd = jnp.dot(d0_ref[...], wd0_ref[...],
                 preferred_element_type=jnp.float32)
    cd += jnp.dot(d1_ref[...], wd1_ref[...],
                  preferred_element_type=jnp.float32)
    cd += jnp.dot(d2_ref[...], wd2_ref[...],
                  preferred_element_type=jnp.float32)
    cd += jnp.dot(d3_ref[...], wd3_ref[...],
                  preferred_element_type=jnp.float32)
    cd += bd_ref[...]
    cs = jnp.dot(s0_ref[...], ws0_ref[...],
                 preferred_element_type=jnp.float32)
    cs += jnp.dot(s1_ref[...], ws1_ref[...],
                  preferred_element_type=jnp.float32)
    cs += bs_ref[...]
    cd_ref[...] = cd
    cs_ref[...] = cs
    logits_ref[...] = b_ref[...] + cd + cs


def _sc_body(logits_hbm, mask_hbm, w_hbm, lg_v, mk_v, wt_v):
    wid = lax.axis_index("s") * _NC + lax.axis_index("c")
    base = wid * _ROWS * E
    pltpu.sync_copy(logits_hbm.at[pl.ds(base, _ROWS * E)], lg_v)

    lane = lax.iota(jnp.int32, _L)
    perm_hi = (lane + 8) & 15

    def merge_top8(a, b):
        b_perm = b.at[perm_hi].get(mode="promise_in_bounds")
        comb = jnp.where(lane < 8, a, b_perm)
        s, _ = plsc.sort_key_val(comb, lane, descending=True)
        return s

    def tok(t, carry):
        v = [lg_v[pl.ds(t * E + _L * i, _L)] for i in range(4)]
        s = [plsc.sort_key_val(v[i], lane, descending=True)[0]
             for i in range(4)]
        fin = merge_top8(merge_top8(s[0], s[1]), merge_top8(s[2], s[3]))
        t8 = fin.at[lane * 0 + (TOP_K - 1)].get(mode="promise_in_bounds")
        m0 = fin.at[lane * 0].get(mode="promise_in_bounds")
        es = []
        for i in range(4):
            ge = v[i] >= t8
            mk_v[pl.ds(t * E + _L * i, _L)] = jnp.where(ge, 1.0, 0.0)
            es.append(jnp.where(ge, jnp.exp(v[i] - m0), 0.0))
        tot = jnp.sum(es[0] + es[1] + es[2] + es[3])
        for i in range(4):
            wt_v[pl.ds(t * E + _L * i, _L)] = es[i] / tot
        return carry

    lax.fori_loop(0, _ROWS, tok, 0)
    pltpu.sync_copy(mk_v, mask_hbm.at[pl.ds(base, _ROWS * E)])
    pltpu.sync_copy(wt_v, w_hbm.at[pl.ds(base, _ROWS * E)])


def _sc_route(logits):
    f = pl.kernel(
        _sc_body,
        out_type=[jax.ShapeDtypeStruct((N * E,), jnp.float32)] * 2,
        mesh=plsc.VectorSubcoreMesh(
            core_axis_name="c", subcore_axis_name="s",
            num_cores=_NC, num_subcores=_NS),
        scratch_types=[pltpu.VMEM((_ROWS * E,), jnp.float32)] * 3,
        compiler_params=pltpu.CompilerParams(needs_layout_passes=False),
    )
    mask, w = f(logits.reshape(N * E))
    return mask.reshape(N, E), w.reshape(N, E)


@jax.jit
def _router(dense, sparse, W_dense, b_dense, W_sparse, b_sparse, bias):
    grid = (N // BN,)
    row_spec = pl.BlockSpec((BN, E), lambda i: (i, 0))
    full = lambda shape: pl.BlockSpec(shape, lambda i: (0, 0))
    KD = D_DENSE // 4
    KS = D_SPARSE // 2
    logits, cd, cs = pl.pallas_call(
        _tc_body,
        grid=grid,
        in_specs=[
            pl.BlockSpec((BN, KD), lambda i: (i, 0)),
            pl.BlockSpec((BN, KD), lambda i: (i, 1)),
            pl.BlockSpec((BN, KD), lambda i: (i, 2)),
            pl.BlockSpec((BN, KD), lambda i: (i, 3)),
            pl.BlockSpec((BN, KS), lambda i: (i, 0)),
            pl.BlockSpec((BN, KS), lambda i: (i, 1)),
            pl.BlockSpec((KD, E), lambda i: (0, 0)),
            pl.BlockSpec((KD, E), lambda i: (1, 0)),
            pl.BlockSpec((KD, E), lambda i: (2, 0)),
            pl.BlockSpec((KD, E), lambda i: (3, 0)),
            full((1, E)),
            pl.BlockSpec((KS, E), lambda i: (0, 0)),
            pl.BlockSpec((KS, E), lambda i: (1, 0)),
            full((1, E)),
            full((1, E)),
        ],
        out_specs=[row_spec] * 3,
        out_shape=[jax.ShapeDtypeStruct((N, E), jnp.float32)] * 3,
    )(dense, dense, dense, dense, sparse, sparse,
      W_dense, W_dense, W_dense, W_dense, b_dense.reshape(1, E),
      W_sparse, W_sparse, b_sparse.reshape(1, E), bias.reshape(1, E))
    mask, weights = _sc_route(logits)
    return logits, weights, mask, cd, cs


def kernel(dense, sparse, W_dense, b_dense, W_sparse, b_sparse, bias):
    logits, weights, topk_mask, c_dense, c_sparse = _router(
        dense, sparse, W_dense, b_dense, W_sparse, b_sparse, bias)
    return (logits, weights, topk_mask, c_dense, c_sparse)


# speedup vs baseline: 1.0115x; 1.0067x over previous
import functools

import jax
import jax.numpy as jnp
from jax import lax
from jax.experimental import pallas as pl
from jax.experimental.pallas import tpu as pltpu
from jax.experimental.pallas import tpu_sc as plsc

N = 16384
D_DENSE = 2048
D_SPARSE = 1024
E = 64
TOP_K = 8
BN = 1024

_NC, _NS, _L = 2, 16, 16
_NW = _NC * _NS
_ROWS = N // _NW


def _tc_body(d0_ref, d1_ref, d2_ref, d3_ref, s0_ref, s1_ref,
             wd0_ref, wd1_ref, wd2_ref, wd3_ref, bd_ref,
             ws0_ref, ws1_ref, bs_ref, b_ref,
             logits_ref, cd_ref, cs_ref):
    cd = jnp.dot(d0_ref[...], wd0_ref[...],
                 preferred_element_type=jnp.float32)
    cd += jnp.dot(d1_ref[...], wd1_ref[...],
                  preferred_element_type=jnp.float32)
    cd += jnp.dot(d2_ref[...], wd2_ref[...],
                  preferred_element_type=jnp.float32)
    cd += jnp.dot(d3_ref[...], wd3_ref[...],
                  preferred_element_type=jnp.float32)
    cd += bd_ref[...]
    cs = jnp.dot(s0_ref[...], ws0_ref[...],
                 preferred_element_type=jnp.float32)
    cs += jnp.dot(s1_ref[...], ws1_ref[...],
                  preferred_element_type=jnp.float32)
    cs += bs_ref[...]
    cd_ref[...] = cd
    cs_ref[...] = cs
    logits_ref[...] = b_ref[...] + cd + cs


def _sc_body(logits_hbm, mask_hbm, w_hbm, lg_v, mk_v, wt_v):
    wid = lax.axis_index("s") * _NC + lax.axis_index("c")
    base = wid * _ROWS * E
    pltpu.sync_copy(logits_hbm.at[pl.ds(base, _ROWS * E)], lg_v)

    lane = lax.iota(jnp.int32, _L)
    perm_hi = (lane + 8) & 15

    def merge_top8(a, b):
        b_perm = b.at[perm_hi].get(mode="promise_in_bounds")
        comb = jnp.where(lane < 8, a, b_perm)
        s, _ = plsc.sort_key_val(comb, lane, descending=True)
        return s

    def tok(t, carry):
        v = [lg_v[pl.ds(t * E + _L * i, _L)] for i in range(4)]
        s = [plsc.sort_key_val(v[i], lane, descending=True)[0]
             for i in range(4)]
        fin = merge_top8(merge_top8(s[0], s[1]), merge_top8(s[2], s[3]))
        t8 = fin.at[lane * 0 + (TOP_K - 1)].get(mode="promise_in_bounds")
        m0 = fin.at[lane * 0].get(mode="promise_in_bounds")
        es = []
        for i in range(4):
            ge = v[i] >= t8
            mk_v[pl.ds(t * E + _L * i, _L)] = jnp.where(ge, 1.0, 0.0)
            es.append(jnp.where(ge, jnp.exp(v[i] - m0), 0.0))
        tot = jnp.sum(es[0] + es[1] + es[2] + es[3])
        for i in range(4):
            wt_v[pl.ds(t * E + _L * i, _L)] = es[i] / tot
        return carry

    lax.fori_loop(0, _ROWS, tok, 0)
    pltpu.sync_copy(mk_v, mask_hbm.at[pl.ds(base, _ROWS * E)])
    pltpu.sync_copy(wt_v, w_hbm.at[pl.ds(base, _ROWS * E)])


def _sc_route(logits):
    f = pl.kernel(
        _sc_body,
        out_type=[jax.ShapeDtypeStruct((N * E,), jnp.float32)] * 2,
        mesh=plsc.VectorSubcoreMesh(
            core_axis_name="c", subcore_axis_name="s",
            num_cores=_NC, num_subcores=_NS),
        scratch_types=[pltpu.VMEM((_ROWS * E,), jnp.float32)] * 3,
        compiler_params=pltpu.CompilerParams(needs_layout_passes=False),
    )
    mask, w = f(logits.reshape(N * E))
    return mask.reshape(N, E), w.reshape(N, E)


@jax.jit
def _router(dense, sparse, W_dense, b_dense, W_sparse, b_sparse, bias):
    grid = (N // BN,)
    row_spec = pl.BlockSpec((BN, E), lambda i: (i, 0))
    full = lambda shape: pl.BlockSpec(shape, lambda i: (0, 0))
    KD = D_DENSE // 4
    KS = D_SPARSE // 2
    logits, cd, cs = pl.pallas_call(
        _tc_body,
        grid=grid,
        in_specs=[
            pl.BlockSpec((BN, KD), lambda i: (i, 0)),
            pl.BlockSpec((BN, KD), lambda i: (i, 1)),
            pl.BlockSpec((BN, KD), lambda i: (i, 2)),
            pl.BlockSpec((BN, KD), lambda i: (i, 3)),
            pl.BlockSpec((BN, KS), lambda i: (i, 0)),
            pl.BlockSpec((BN, KS), lambda i: (i, 1)),
            pl.BlockSpec((KD, E), lambda i: (0, 0)),
            pl.BlockSpec((KD, E), lambda i: (1, 0)),
            pl.BlockSpec((KD, E), lambda i: (2, 0)),
            pl.BlockSpec((KD, E), lambda i: (3, 0)),
            full((1, E)),
            pl.BlockSpec((KS, E), lambda i: (0, 0)),
            pl.BlockSpec((KS, E), lambda i: (1, 0)),
            full((1, E)),
            full((1, E)),
        ],
        out_specs=[row_spec] * 3,
        out_shape=[jax.ShapeDtypeStruct((N, E), jnp.float32)] * 3,
    )(dense, dense, dense, dense, sparse, sparse,
      W_dense, W_dense, W_dense, W_dense, b_dense.reshape(1, E),
      W_sparse, W_sparse, b_sparse.reshape(1, E), bias.reshape(1, E))
    mask, weights = _sc_route(logits)
    return logits, weights, mask, cd, cs


def kernel(dense, sparse, W_dense, b_dense, W_sparse, b_sparse, bias):
    logits, weights, topk_mask, c_dense, c_sparse = _router(
        dense, sparse, W_dense, b_dense, W_sparse, b_sparse, bias)
    return (logits, weights, topk_mask, c_dense, c_sparse)


# speedup vs baseline: 1.4439x; 1.4274x over previous
import functools

import jax
import jax.numpy as jnp
from jax import lax
from jax.experimental import pallas as pl
from jax.experimental.pallas import tpu as pltpu
from jax.experimental.pallas import tpu_sc as plsc

N = 16384
D_DENSE = 2048
D_SPARSE = 1024
E = 64
TOP_K = 8
BN = 1024

_NC, _NS, _L = 2, 16, 16
_NW = _NC * _NS
_TOK = N // _NW


def _tc_body(d0_ref, d1_ref, d2_ref, d3_ref, s0_ref, s1_ref,
             wd0_ref, wd1_ref, wd2_ref, wd3_ref, bd_ref,
             ws0_ref, ws1_ref, bs_ref, b_ref,
             logits_ref, cd_ref, cs_ref):
    cd = jnp.dot(d0_ref[...], wd0_ref[...],
                 preferred_element_type=jnp.float32)
    cd += jnp.dot(d1_ref[...], wd1_ref[...],
                  preferred_element_type=jnp.float32)
    cd += jnp.dot(d2_ref[...], wd2_ref[...],
                  preferred_element_type=jnp.float32)
    cd += jnp.dot(d3_ref[...], wd3_ref[...],
                  preferred_element_type=jnp.float32)
    cd += bd_ref[...]
    cs = jnp.dot(s0_ref[...], ws0_ref[...],
                 preferred_element_type=jnp.float32)
    cs += jnp.dot(s1_ref[...], ws1_ref[...],
                  preferred_element_type=jnp.float32)
    cs += bs_ref[...]
    cd_ref[...] = cd.T
    cs_ref[...] = cs.T
    logits_ref[...] = (b_ref[...] + cd + cs).T


def _sorted2(a, b):
    return jnp.maximum(a, b), jnp.minimum(a, b)


def _merge_sorted(a, b):
    n = len(a)
    seq = list(a) + list(reversed(b))
    return _bitonic_sort(seq)


def _bitonic_sort(seq):
    n = len(seq)
    d = n // 2
    while d >= 1:
        for i in range(0, n, 2 * d):
            for j in range(i, i + d):
                hi, lo = _sorted2(seq[j], seq[j + d])
                seq[j], seq[j + d] = hi, lo
        d //= 2
    return seq


def _top8_of_two_sorted8(a, b):
    return [jnp.maximum(a[i], b[7 - i]) for i in range(8)]


def _sc_body(logits_hbm, mask_hbm, w_hbm, lg_v, mk_v, wt_v):
    wid = lax.axis_index("s") * _NC + lax.axis_index("c")
    base = wid * _TOK
    pltpu.sync_copy(logits_hbm.at[:, pl.ds(base, _TOK)], lg_v)

    def group(g, carry):
        col = g * _L
        v = [lg_v[e, pl.ds(col, _L)] for e in range(E)]
        s2 = [_sorted2(v[2 * i], v[2 * i + 1]) for i in range(32)]
        s4 = [_merge_sorted(s2[2 * i], s2[2 * i + 1]) for i in range(16)]
        s8 = [_merge_sorted(s4[2 * i], s4[2 * i + 1]) for i in range(8)]
        t4 = [_bitonic_sort(_top8_of_two_sorted8(s8[2 * i], s8[2 * i + 1]))
              for i in range(4)]
        t2 = [_bitonic_sort(_top8_of_two_sorted8(t4[2 * i], t4[2 * i + 1]))
              for i in range(2)]
        top = _top8_of_two_sorted8(t2[0], t2[1])
        t8 = top[0]
        m0 = top[0]
        for r in top[1:]:
            t8 = jnp.minimum(t8, r)
            m0 = jnp.maximum(m0, r)
        es = []
        tot = None
        for e in range(E):
            ge = v[e] >= t8
            mk_v[e, pl.ds(col, _L)] = jnp.where(ge, 1.0, 0.0)
            ee = jnp.where(ge, jnp.exp(v[e] - m0), 0.0)
            es.append(ee)
            tot = ee if tot is None else tot + ee
        inv = 1.0 / tot
        for e in range(E):
            wt_v[e, pl.ds(col, _L)] = es[e] * inv
        return carry

    lax.fori_loop(0, _TOK // _L, group, 0)
    pltpu.sync_copy(mk_v, mask_hbm.at[:, pl.ds(base, _TOK)])
    pltpu.sync_copy(wt_v, w_hbm.at[:, pl.ds(base, _TOK)])


def _sc_route(logits_t):
    f = pl.kernel(
        _sc_body,
        out_type=[jax.ShapeDtypeStruct((E, N), jnp.float32)] * 2,
        mesh=plsc.VectorSubcoreMesh(
            core_axis_name="c", subcore_axis_name="s",
            num_cores=_NC, num_subcores=_NS),
        scratch_types=[pltpu.VMEM((E, _TOK), jnp.float32)] * 3,
        compiler_params=pltpu.CompilerParams(
            needs_layout_passes=False, use_tc_tiling_on_sc=True),
    )
    return f(logits_t)


def _router(dense, sparse, W_dense, b_dense, W_sparse, b_sparse, bias):
    grid = (N // BN,)
    col_spec = pl.BlockSpec((E, BN), lambda i: (0, i))
    full = lambda shape: pl.BlockSpec(shape, lambda i: (0, 0))
    KD = D_DENSE // 4
    KS = D_SPARSE // 2
    logits_t, cd_t, cs_t = pl.pallas_call(
        _tc_body,
        grid=grid,
        in_specs=[
            pl.BlockSpec((BN, KD), lambda i: (i, 0)),
            pl.BlockSpec((BN, KD), lambda i: (i, 1)),
            pl.BlockSpec((BN, KD), lambda i: (i, 2)),
            pl.BlockSpec((BN, KD), lambda i: (i, 3)),
            pl.BlockSpec((BN, KS), lambda i: (i, 0)),
            pl.BlockSpec((BN, KS), lambda i: (i, 1)),
            pl.BlockSpec((KD, E), lambda i: (0, 0)),
            pl.BlockSpec((KD, E), lambda i: (1, 0)),
            pl.BlockSpec((KD, E), lambda i: (2, 0)),
            pl.BlockSpec((KD, E), lambda i: (3, 0)),
            full((1, E)),
            pl.BlockSpec((KS, E), lambda i: (0, 0)),
            pl.BlockSpec((KS, E), lambda i: (1, 0)),
            full((1, E)),
            full((1, E)),
        ],
        out_specs=[col_spec] * 3,
        out_shape=[jax.ShapeDtypeStruct((E, N), jnp.float32)] * 3,
    )(dense, dense, dense, dense, sparse, sparse,
      W_dense, W_dense, W_dense, W_dense, b_dense.reshape(1, E),
      W_sparse, W_sparse, b_sparse.reshape(1, E), bias.reshape(1, E))
    mask_t, w_t = _sc_route(logits_t)
    return (jnp.swapaxes(logits_t, 0, 1), jnp.swapaxes(w_t, 0, 1),
            jnp.swapaxes(mask_t, 0, 1), jnp.swapaxes(cd_t, 0, 1),
            jnp.swapaxes(cs_t, 0, 1))


def kernel(dense, sparse, W_dense, b_dense, W_sparse, b_sparse, bias):
    return _router(dense, sparse, W_dense, b_dense, W_sparse, b_sparse, bias)


# speedup vs baseline: 1.4486x; 1.0032x over previous
import functools

import jax
import jax.numpy as jnp
from jax import lax
from jax.experimental import pallas as pl
from jax.experimental.pallas import tpu as pltpu
from jax.experimental.pallas import tpu_sc as plsc

N = 16384
D_DENSE = 2048
D_SPARSE = 1024
E = 64
TOP_K = 8
BN = 1024

_NC, _NS, _L = 2, 16, 16
_NW = _NC * _NS
_TOK = N // _NW


def _tc_body(d0_ref, d1_ref, d2_ref, d3_ref, s0_ref, s1_ref,
             wd0_ref, wd1_ref, wd2_ref, wd3_ref, bd_ref,
             ws0_ref, ws1_ref, bs_ref, b_ref,
             logits_ref, cd_ref, cs_ref):
    cd = jnp.dot(d0_ref[...], wd0_ref[...],
                 preferred_element_type=jnp.float32)
    cd += jnp.dot(d1_ref[...], wd1_ref[...],
                  preferred_element_type=jnp.float32)
    cd += jnp.dot(d2_ref[...], wd2_ref[...],
                  preferred_element_type=jnp.float32)
    cd += jnp.dot(d3_ref[...], wd3_ref[...],
                  preferred_element_type=jnp.float32)
    cd += bd_ref[...]
    cs = jnp.dot(s0_ref[...], ws0_ref[...],
                 preferred_element_type=jnp.float32)
    cs += jnp.dot(s1_ref[...], ws1_ref[...],
                  preferred_element_type=jnp.float32)
    cs += bs_ref[...]
    cd_ref[...] = cd.T
    cs_ref[...] = cs.T
    logits_ref[...] = (b_ref[...] + cd + cs).T


def _sorted2(a, b):
    return jnp.maximum(a, b), jnp.minimum(a, b)


def _merge_sorted(a, b):
    n = len(a)
    seq = list(a) + list(reversed(b))
    return _bitonic_sort(seq)


def _bitonic_sort(seq):
    n = len(seq)
    d = n // 2
    while d >= 1:
        for i in range(0, n, 2 * d):
            for j in range(i, i + d):
                hi, lo = _sorted2(seq[j], seq[j + d])
                seq[j], seq[j + d] = hi, lo
        d //= 2
    return seq


def _top8_of_two_sorted8(a, b):
    return [jnp.maximum(a[i], b[7 - i]) for i in range(8)]


def _sc_body(logits_hbm, mask_hbm, w_hbm, lg_v, mk_v, wt_v,
             s_in0, s_in1, s_mk0, s_wt0):
    wid = lax.axis_index("s") * _NC + lax.axis_index("c")
    base = wid * _TOK
    H = _TOK // 2
    in0 = pltpu.async_copy(logits_hbm.at[:, pl.ds(base, H)],
                           lg_v.at[:, pl.ds(0, H)], s_in0)
    in1 = pltpu.async_copy(logits_hbm.at[:, pl.ds(base + H, H)],
                           lg_v.at[:, pl.ds(H, H)], s_in1)

    def group(g, carry):
        col = g * _L
        v = [lg_v[e, pl.ds(col, _L)] for e in range(E)]
        s2 = [_sorted2(v[2 * i], v[2 * i + 1]) for i in range(32)]
        s4 = [_merge_sorted(s2[2 * i], s2[2 * i + 1]) for i in range(16)]
        s8 = [_merge_sorted(s4[2 * i], s4[2 * i + 1]) for i in range(8)]
        t4 = [_bitonic_sort(_top8_of_two_sorted8(s8[2 * i], s8[2 * i + 1]))
              for i in range(4)]
        t2 = [_bitonic_sort(_top8_of_two_sorted8(t4[2 * i], t4[2 * i + 1]))
              for i in range(2)]
        top = _top8_of_two_sorted8(t2[0], t2[1])
        t8 = top[0]
        m0 = top[0]
        for r in top[1:]:
            t8 = jnp.minimum(t8, r)
            m0 = jnp.maximum(m0, r)
        es = []
        tot = None
        for e in range(E):
            ge = v[e] >= t8
            mk_v[e, pl.ds(col, _L)] = jnp.where(ge, 1.0, 0.0)
            ee = jnp.where(ge, jnp.exp(v[e] - m0), 0.0)
            es.append(ee)
            tot = ee if tot is None else tot + ee
        inv = 1.0 / tot
        for e in range(E):
            wt_v[e, pl.ds(col, _L)] = es[e] * inv
        return carry

    in0.wait()
    lax.fori_loop(0, H // _L, group, 0)
    mk0 = pltpu.async_copy(mk_v.at[:, pl.ds(0, H)],
                           mask_hbm.at[:, pl.ds(base, H)], s_mk0)
    wt0 = pltpu.async_copy(wt_v.at[:, pl.ds(0, H)],
                           w_hbm.at[:, pl.ds(base, H)], s_wt0)
    in1.wait()
    lax.fori_loop(H // _L, _TOK // _L, group, 0)
    pltpu.sync_copy(mk_v.at[:, pl.ds(H, H)],
                    mask_hbm.at[:, pl.ds(base + H, H)])
    pltpu.sync_copy(wt_v.at[:, pl.ds(H, H)],
                    w_hbm.at[:, pl.ds(base + H, H)])
    mk0.wait()
    wt0.wait()


def _sc_route(logits_t):
    f = pl.kernel(
        _sc_body,
        out_type=[jax.ShapeDtypeStruct((E, N), jnp.float32)] * 2,
        mesh=plsc.VectorSubcoreMesh(
            core_axis_name="c", subcore_axis_name="s",
            num_cores=_NC, num_subcores=_NS),
        scratch_types=[pltpu.VMEM((E, _TOK), jnp.float32)] * 3 +
                      [pltpu.SemaphoreType.DMA] * 4,
        compiler_params=pltpu.CompilerParams(
            needs_layout_passes=False, use_tc_tiling_on_sc=True),
    )
    return f(logits_t)


def _router(dense, sparse, W_dense, b_dense, W_sparse, b_sparse, bias):
    grid = (N // BN,)
    col_spec = pl.BlockSpec((E, BN), lambda i: (0, i))
    full = lambda shape: pl.BlockSpec(shape, lambda i: (0, 0))
    KD = D_DENSE // 4
    KS = D_SPARSE // 2
    logits_t, cd_t, cs_t = pl.pallas_call(
        _tc_body,
        grid=grid,
        in_specs=[
            pl.BlockSpec((BN, KD), lambda i: (i, 0)),
            pl.BlockSpec((BN, KD), lambda i: (i, 1)),
            pl.BlockSpec((BN, KD), lambda i: (i, 2)),
            pl.BlockSpec((BN, KD), lambda i: (i, 3)),
            pl.BlockSpec((BN, KS), lambda i: (i, 0)),
            pl.BlockSpec((BN, KS), lambda i: (i, 1)),
            pl.BlockSpec((KD, E), lambda i: (0, 0)),
            pl.BlockSpec((KD, E), lambda i: (1, 0)),
            pl.BlockSpec((KD, E), lambda i: (2, 0)),
            pl.BlockSpec((KD, E), lambda i: (3, 0)),
            full((1, E)),
            pl.BlockSpec((KS, E), lambda i: (0, 0)),
            pl.BlockSpec((KS, E), lambda i: (1, 0)),
            full((1, E)),
            full((1, E)),
        ],
        out_specs=[col_spec] * 3,
        out_shape=[jax.ShapeDtypeStruct((E, N), jnp.float32)] * 3,
    )(dense, dense, dense, dense, sparse, sparse,
      W_dense, W_dense, W_dense, W_dense, b_dense.reshape(1, E),
      W_sparse, W_sparse, b_sparse.reshape(1, E), bias.reshape(1, E))
    mask_t, w_t = _sc_route(logits_t)
    return (jnp.swapaxes(logits_t, 0, 1), jnp.swapaxes(w_t, 0, 1),
            jnp.swapaxes(mask_t, 0, 1), jnp.swapaxes(cd_t, 0, 1),
            jnp.swapaxes(cs_t, 0, 1))


def kernel(dense, sparse, W_dense, b_dense, W_sparse, b_sparse, bias):
    return _router(dense, sparse, W_dense, b_dense, W_sparse, b_sparse, bias)
